# baseline (device time: 23904 ns/iter reference)
import jax
import jax.numpy as jnp
from jax import lax
from jax.experimental import pallas as pl
from jax.experimental.pallas import tpu as pltpu

N_DEV = 16
D = 512
ROWS = D // N_DEV
HALVES = 2
HR = ROWS // HALVES


def kernel(partial, resid, gamma):
    def body(x_ref, resid_ref, gamma_ref, out_ref,
             stage_ref, recv_ref, gather_ref,
             send_sems, recv_sems, send_sems2, recv_sems2):
        my = lax.axis_index("i")

        barrier_sem = pltpu.get_barrier_semaphore()
        for j in range(N_DEV):
            @pl.when(my != j)
            def _(j_=j):
                pl.semaphore_signal(
                    barrier_sem, inc=1,
                    device_id=(j_,), device_id_type=pl.DeviceIdType.MESH,
                )
        pl.semaphore_wait(barrier_sem, N_DEV - 1)

        stage_ref[...] = x_ref[0].astype(jnp.bfloat16).reshape(
            N_DEV, HALVES, HR, D)

        def rs_rdma(h, j, slot):
            return pltpu.make_async_remote_copy(
                src_ref=stage_ref.at[j, h],
                dst_ref=recv_ref.at[slot, h],
                send_sem=send_sems.at[h, j],
                recv_sem=recv_sems.at[h, slot],
                device_id=(j,),
                device_id_type=pl.DeviceIdType.MESH,
            )

        for h in range(HALVES):
            for j in range(N_DEV):
                @pl.when(my != j)
                def _(h_=h, j_=j):
                    rs_rdma(h_, j_, my).start()

        recv_ref[pl.ds(my, 1)] = stage_ref[pl.ds(my, 1)]

        def ag_rdma(h, j, slot):
            return pltpu.make_async_remote_copy(
                src_ref=gather_ref.at[slot, h],
                dst_ref=gather_ref.at[slot, h],
                send_sem=send_sems2.at[h, j],
                recv_sem=recv_sems2.at[h, slot],
                device_id=(j,),
                device_id_type=pl.DeviceIdType.MESH,
            )

        for h in range(HALVES):
            for s in range(N_DEV):
                @pl.when(my != s)
                def _(h_=h, s_=s):
                    rs_rdma(h_, s_, s_).wait_recv()

            acc = jnp.sum(recv_ref[:, h].astype(jnp.float32), axis=0)
            y = acc + resid_ref[pl.ds(my * ROWS + h * HR, HR), :]
            ms = jnp.mean(y * y, axis=-1, keepdims=True)
            out_chunk = y * lax.rsqrt(ms + 1e-6) * gamma_ref[:][None, :]
            gather_ref[pl.ds(my, 1), pl.ds(h, 1)] = (
                out_chunk.astype(jnp.bfloat16)[None, None])

            for j in range(N_DEV):
                @pl.when(my != j)
                def _(h_=h, j_=j):
                    ag_rdma(h_, j_, my).start()

        for h in range(HALVES):
            for j in range(N_DEV):
                @pl.when(my != j)
                def _(h_=h, j_=j):
                    rs_rdma(h_, j_, my).wait_send()
        for h in range(HALVES):
            for s in range(N_DEV):
                @pl.when(my != s)
                def _(h_=h, s_=s):
                    ag_rdma(h_, s_, s_).wait_recv()
        for h in range(HALVES):
            for j in range(N_DEV):
                @pl.when(my != j)
                def _(h_=h, j_=j):
                    ag_rdma(h_, j_, my).wait_send()

        out_ref[...] = gather_ref[...].astype(jnp.float32).reshape(D, D)

    return pl.pallas_call(
        body,
        out_shape=jax.ShapeDtypeStruct((D, D), jnp.float32),
        in_specs=[
            pl.BlockSpec(memory_space=pltpu.VMEM),
            pl.BlockSpec(memory_space=pltpu.VMEM),
            pl.BlockSpec(memory_space=pltpu.VMEM),
        ],
        out_specs=pl.BlockSpec(memory_space=pltpu.VMEM),
        scratch_shapes=[
            pltpu.VMEM((N_DEV, HALVES, HR, D), jnp.bfloat16),
            pltpu.VMEM((N_DEV, HALVES, HR, D), jnp.bfloat16),
            pltpu.VMEM((N_DEV, HALVES, HR, D), jnp.bfloat16),
            pltpu.SemaphoreType.DMA((HALVES, N_DEV)),
            pltpu.SemaphoreType.DMA((HALVES, N_DEV)),
            pltpu.SemaphoreType.DMA((HALVES, N_DEV)),
            pltpu.SemaphoreType.DMA((HALVES, N_DEV)),
        ],
        compiler_params=pltpu.CompilerParams(collective_id=0),
    )(partial, resid, gamma)


# device time: 21086 ns/iter; 1.1336x vs baseline; 1.1336x over previous
import jax
import jax.numpy as jnp
from jax import lax
from jax.experimental import pallas as pl
from jax.experimental.pallas import tpu as pltpu

N_DEV = 16
D = 512
ROWS = D // N_DEV


def kernel(partial, resid, gamma):
    def body(x_ref, resid_ref, gamma_ref, out_ref,
             stage_ref, recv_ref, gather_ref,
             send_sems, recv_sems, send_sems2, recv_sems2):
        my = lax.axis_index("i")

        stage_ref[...] = x_ref[0].astype(jnp.bfloat16).reshape(N_DEV, ROWS, D)

        def rs_rdma(d):
            j = lax.rem(my + d, N_DEV)
            return pltpu.make_async_remote_copy(
                src_ref=stage_ref.at[j],
                dst_ref=recv_ref.at[my],
                send_sem=send_sems.at[d],
                recv_sem=recv_sems.at[my],
                device_id=(j,),
                device_id_type=pl.DeviceIdType.MESH,
            )

        def rs_wait(d):
            s = lax.rem(my + d, N_DEV)
            return pltpu.make_async_remote_copy(
                src_ref=stage_ref.at[s],
                dst_ref=recv_ref.at[s],
                send_sem=send_sems.at[d],
                recv_sem=recv_sems.at[s],
                device_id=(s,),
                device_id_type=pl.DeviceIdType.MESH,
            ), s

        for d in range(1, N_DEV):
            rs_rdma(d).start()

        acc = x_ref[0, pl.ds(my * ROWS, ROWS), :]
        for d in range(1, N_DEV):
            w, s = rs_wait(d)
            w.wait_recv()
            acc = acc + recv_ref[pl.ds(s, 1)][0].astype(jnp.float32)

        y = acc + resid_ref[pl.ds(my * ROWS, ROWS), :]
        ms = jnp.mean(y * y, axis=-1, keepdims=True)
        out_chunk = y * lax.rsqrt(ms + 1e-6) * gamma_ref[:][None, :]
        out_ref[pl.ds(my * ROWS, ROWS), :] = out_chunk
        gather_ref[pl.ds(my, 1)] = out_chunk.astype(jnp.bfloat16)[None]

        def ag_rdma(d):
            j = lax.rem(my + d, N_DEV)
            return pltpu.make_async_remote_copy(
                src_ref=gather_ref.at[my],
                dst_ref=gather_ref.at[my],
                send_sem=send_sems2.at[d],
                recv_sem=recv_sems2.at[my],
                device_id=(j,),
                device_id_type=pl.DeviceIdType.MESH,
            )

        def ag_wait(d):
            s = lax.rem(my + d, N_DEV)
            return pltpu.make_async_remote_copy(
                src_ref=gather_ref.at[s],
                dst_ref=gather_ref.at[s],
                send_sem=send_sems2.at[d],
                recv_sem=recv_sems2.at[s],
                device_id=(s,),
                device_id_type=pl.DeviceIdType.MESH,
            ), s

        for d in range(1, N_DEV):
            ag_rdma(d).start()

        for d in range(1, N_DEV):
            rs_rdma(d).wait_send()

        for d in range(1, N_DEV):
            w, s = ag_wait(d)
            w.wait_recv()
            out_ref[pl.ds(s * ROWS, ROWS), :] = (
                recv_chunk := gather_ref[pl.ds(s, 1)][0].astype(jnp.float32))

        for d in range(1, N_DEV):
            ag_rdma(d).wait_send()

    return pl.pallas_call(
        body,
        out_shape=jax.ShapeDtypeStruct((D, D), jnp.float32),
        in_specs=[
            pl.BlockSpec(memory_space=pltpu.VMEM),
            pl.BlockSpec(memory_space=pltpu.VMEM),
            pl.BlockSpec(memory_space=pltpu.VMEM),
        ],
        out_specs=pl.BlockSpec(memory_space=pltpu.VMEM),
        scratch_shapes=[
            pltpu.VMEM((N_DEV, ROWS, D), jnp.bfloat16),
            pltpu.VMEM((N_DEV, ROWS, D), jnp.bfloat16),
            pltpu.VMEM((N_DEV, ROWS, D), jnp.bfloat16),
            pltpu.SemaphoreType.DMA((N_DEV,)),
            pltpu.SemaphoreType.DMA((N_DEV,)),
            pltpu.SemaphoreType.DMA((N_DEV,)),
            pltpu.SemaphoreType.DMA((N_DEV,)),
        ],
        compiler_params=pltpu.CompilerParams(skip_device_barrier=True),
    )(partial, resid, gamma)


# device time: 21023 ns/iter; 1.1370x vs baseline; 1.0030x over previous
import jax
import jax.numpy as jnp
from jax import lax
from jax.experimental import pallas as pl
from jax.experimental.pallas import tpu as pltpu

N_DEV = 16
D = 512
ROWS = D // N_DEV


def kernel(partial, resid, gamma):
    def body(x_ref, resid_ref, gamma_ref, out_ref,
             stage_ref, recv_ref, gather_ref,
             send_sems, recv_sems, send_sems2, recv_sems2):
        my = lax.axis_index("i")

        def rs_rdma(d):
            j = lax.rem(my + d, N_DEV)
            return pltpu.make_async_remote_copy(
                src_ref=stage_ref.at[j],
                dst_ref=recv_ref.at[my],
                send_sem=send_sems.at[d],
                recv_sem=recv_sems.at[my],
                device_id=(j,),
                device_id_type=pl.DeviceIdType.MESH,
            )

        def rs_wait(d):
            s = lax.rem(my + d, N_DEV)
            return pltpu.make_async_remote_copy(
                src_ref=stage_ref.at[s],
                dst_ref=recv_ref.at[s],
                send_sem=send_sems.at[d],
                recv_sem=recv_sems.at[s],
                device_id=(s,),
                device_id_type=pl.DeviceIdType.MESH,
            ), s

        for d in range(1, N_DEV):
            j = lax.rem(my + d, N_DEV)
            stage_ref[pl.ds(j, 1)] = (
                x_ref[pl.ds(0, 1), pl.ds(j * ROWS, ROWS), :]
                .astype(jnp.bfloat16).reshape(1, ROWS, D))
            rs_rdma(d).start()

        acc = x_ref[0, pl.ds(my * ROWS, ROWS), :]
        for d in range(1, N_DEV):
            w, s = rs_wait(d)
            w.wait_recv()
            acc = acc + recv_ref[pl.ds(s, 1)][0].astype(jnp.float32)

        y = acc + resid_ref[pl.ds(my * ROWS, ROWS), :]
        ms = jnp.mean(y * y, axis=-1, keepdims=True)
        out_chunk = y * lax.rsqrt(ms + 1e-6) * gamma_ref[:][None, :]
        out_ref[pl.ds(my * ROWS, ROWS), :] = out_chunk
        gather_ref[pl.ds(my, 1)] = out_chunk.astype(jnp.bfloat16)[None]

        def ag_rdma(d):
            j = lax.rem(my + d, N_DEV)
            return pltpu.make_async_remote_copy(
                src_ref=gather_ref.at[my],
                dst_ref=gather_ref.at[my],
                send_sem=send_sems2.at[d],
                recv_sem=recv_sems2.at[my],
                device_id=(j,),
                device_id_type=pl.DeviceIdType.MESH,
            )

        def ag_wait(d):
            s = lax.rem(my + d, N_DEV)
            return pltpu.make_async_remote_copy(
                src_ref=gather_ref.at[s],
                dst_ref=gather_ref.at[s],
                send_sem=send_sems2.at[d],
                recv_sem=recv_sems2.at[s],
                device_id=(s,),
                device_id_type=pl.DeviceIdType.MESH,
            ), s

        for d in range(1, N_DEV):
            ag_rdma(d).start()

        for d in range(1, N_DEV):
            rs_rdma(d).wait_send()

        for d in range(1, N_DEV):
            w, s = ag_wait(d)
            w.wait_recv()
            out_ref[pl.ds(s * ROWS, ROWS), :] = (
                gather_ref[pl.ds(s, 1)][0].astype(jnp.float32))

        for d in range(1, N_DEV):
            ag_rdma(d).wait_send()

    return pl.pallas_call(
        body,
        out_shape=jax.ShapeDtypeStruct((D, D), jnp.float32),
        in_specs=[
            pl.BlockSpec(memory_space=pltpu.VMEM),
            pl.BlockSpec(memory_space=pltpu.VMEM),
            pl.BlockSpec(memory_space=pltpu.VMEM),
        ],
        out_specs=pl.BlockSpec(memory_space=pltpu.VMEM),
        scratch_shapes=[
            pltpu.VMEM((N_DEV, ROWS, D), jnp.bfloat16),
            pltpu.VMEM((N_DEV, ROWS, D), jnp.bfloat16),
            pltpu.VMEM((N_DEV, ROWS, D), jnp.bfloat16),
            pltpu.SemaphoreType.DMA((N_DEV,)),
            pltpu.SemaphoreType.DMA((N_DEV,)),
            pltpu.SemaphoreType.DMA((N_DEV,)),
            pltpu.SemaphoreType.DMA((N_DEV,)),
        ],
        compiler_params=pltpu.CompilerParams(skip_device_barrier=True),
    )(partial, resid, gamma)
